# baseline (speedup 1.0000x reference)
import functools

import jax
import jax.numpy as jnp
from jax import lax
from jax.experimental import pallas as pl
from jax.experimental.pallas import tpu as pltpu
from jax.experimental.pallas import tpu_sc as plsc

H = 128
NC = 2
NS = 16
CUT = 8.0


def _silu(x):
    return 0.5 * x * (1.0 + jnp.tanh(0.5 * x))


def _b16(x):
    return x.astype(jnp.bfloat16)


def _sc_mesh():
    return plsc.VectorSubcoreMesh(
        core_axis_name="c", subcore_axis_name="s", num_cores=NC, num_subcores=NS
    )


def _sc_gather(table, idx):
    E = idx.shape[0]
    NW = NC * NS
    per_w = E // NW
    CH = 80
    n_ch = per_w // CH
    GPS = 5
    SUP = CH * GPS
    n_sup = per_w // SUP

    idx3 = idx.reshape(NW, n_ch, CH)

    @functools.partial(
        pl.kernel,
        out_type=jax.ShapeDtypeStruct((E, H), jnp.float32),
        mesh=_sc_mesh(),
        scratch_types=[
            pltpu.VMEM((n_ch, CH), jnp.int32),
            pltpu.VMEM((SUP, H), jnp.float32),
            pltpu.SemaphoreType.DMA,
            pltpu.SemaphoreType.DMA,
        ],
    )
    def k(table_hbm, idx_hbm, out_hbm, idx_v, rows_v, sem_i, sem_g):
        wid = lax.axis_index("s") * NC + lax.axis_index("c")
        base = wid * per_w
        pltpu.async_copy(idx_hbm.at[wid], idx_v, sem_i).wait()

        @pl.loop(0, n_sup)
        def _(j):
            descs = [
                pltpu.async_copy(
                    table_hbm.at[idx_v.at[j * GPS + kk]],
                    rows_v.at[pl.ds(kk * CH, CH)],
                    sem_g,
                )
                for kk in range(GPS)
            ]
            for d in descs:
                d.wait()
            pltpu.sync_copy(rows_v, out_hbm.at[pl.ds(base + j * SUP, SUP)])

    return k(table, idx3)


def _sc_segsum(payload, dst, n_nodes, init):
    E = dst.shape[0]
    per_s = E // NS
    n_sup = 125
    GPS = 2
    SUP = per_s // n_sup
    CH = SUP // GPS
    assert CH % 8 == 0 and CH <= 128 and SUP * n_sup == per_s
    n_pair = n_sup // 2
    rows_per_s = (n_nodes // NS) // 8 * 8
    tail_rows = n_nodes - (NS - 1) * rows_per_s - rows_per_s
    tail_off = NS * rows_per_s

    dst4 = dst.reshape(NS, n_sup, GPS, CH)

    @functools.partial(
        pl.kernel,
        out_type=jax.ShapeDtypeStruct((NC, n_nodes, H), jnp.float32),
        mesh=_sc_mesh(),
        scratch_types=[
            pltpu.VMEM((GPS, CH), jnp.int32),
            pltpu.VMEM((GPS, CH), jnp.int32),
            pltpu.VMEM((SUP, H), jnp.float32),
            pltpu.VMEM((SUP, H), jnp.float32),
            pltpu.VMEM_SHARED((n_nodes, H), jnp.float32),
            pltpu.SemaphoreType.DMA,
            pltpu.SemaphoreType.DMA,
            pltpu.SemaphoreType.DMA,
            pltpu.SemaphoreType.DMA,
            pltpu.SemaphoreType.DMA,
        ],
    )
    def k(pay_hbm, dst_hbm, init_hbm, out_hbm, ib0, ib1, rb0, rb1, acc_sh,
          si0, si1, sp0, sp1, sa):
        c = lax.axis_index("c")
        s = lax.axis_index("s")
        idxb, rowsb, sib, spb = (ib0, ib1), (rb0, rb1), (si0, si1), (sp0, sp1)
        base = s * per_s

        def start(j, b):
            pltpu.async_copy(dst_hbm.at[s, j], idxb[b], sib[b])
            pltpu.async_copy(
                pay_hbm.at[c, pl.ds(base + j * SUP, SUP)], rowsb[b], spb[b]
            )

        def work(b):
            pltpu.make_async_copy(dst_hbm.at[s, 0], idxb[b], sib[b]).wait()
            pltpu.make_async_copy(
                pay_hbm.at[c, pl.ds(base, SUP)], rowsb[b], spb[b]
            ).wait()
            descs = [
                pltpu.async_copy(
                    rowsb[b].at[pl.ds(kk * CH, CH)],
                    acc_sh.at[idxb[b].at[kk]],
                    sa,
                    add=True,
                )
                for kk in range(GPS)
            ]
            for d in descs:
                d.wait()

        start(0, 0)
        r0 = s * rows_per_s
        pltpu.sync_copy(
            init_hbm.at[c, pl.ds(r0, rows_per_s)],
            acc_sh.at[pl.ds(r0, rows_per_s)],
        )

        @pl.when(s == NS - 1)
        def _():
            pltpu.sync_copy(
                init_hbm.at[c, pl.ds(tail_off, tail_rows)],
                acc_sh.at[pl.ds(tail_off, tail_rows)],
            )

        plsc.subcore_barrier()

        @pl.loop(0, n_pair)
        def _(i):
            start(2 * i + 1, 1)
            work(0)
            start(2 * i + 2, 0)
            work(1)

        work(0)

        plsc.subcore_barrier()
        pltpu.sync_copy(
            acc_sh.at[pl.ds(r0, rows_per_s)], out_hbm.at[c, pl.ds(r0, rows_per_s)]
        )

        @pl.when(s == NS - 1)
        def _():
            pltpu.sync_copy(
                acc_sh.at[pl.ds(tail_off, tail_rows)],
                out_hbm.at[c, pl.ds(tail_off, tail_rows)],
            )

    return k(payload, dst4, init)


def _tc_node_proj(x_s, w, b):
    n = x_s.shape[0]
    BN = 2000

    def body(x_ref, w_ref, b_ref, o_ref):
        o_ref[...] = (
            jnp.dot(x_ref[...], w_ref[...], preferred_element_type=jnp.float32)
            + b_ref[...]
        )

    return pl.pallas_call(
        body,
        grid=(n // BN,),
        in_specs=[
            pl.BlockSpec((BN, H), lambda i: (i, 0)),
            pl.BlockSpec((H, H), lambda i: (0, 0)),
            pl.BlockSpec((1, H), lambda i: (0, 0)),
        ],
        out_specs=pl.BlockSpec((BN, H), lambda i: (i, 0)),
        out_shape=jax.ShapeDtypeStruct((n, H), jnp.float32),
    )(x_s, w, b.reshape(1, H))


BE = 3200


def _full(shape):
    return pl.BlockSpec(shape, lambda *_: tuple(0 for _ in shape))


def _d12(sp, to, sw1_ref, sw2_ref, tw1_ref, tw2_ref):
    dot = functools.partial(jnp.dot, preferred_element_type=jnp.float32)
    d1 = dot(_b16(dot(sp, sw1_ref[...])), sw2_ref[...])
    d2 = dot(_b16(dot(to, tw1_ref[...])), tw2_ref[...])
    return d1, d2


def _tc_fji(edge_attr, sphe16, tors16, p):
    E = edge_attr.shape[0]
    sphe_d, tors_d = sphe16.shape[1], tors16.shape[1]
    mid = p["sphe_w1"].shape[1]

    def body(ea_ref, sp_ref, to_ref, sw1_ref, sw2_ref, tw1_ref, tw2_ref,
             fpw_ref, fpb_ref, cfa_ref, cfb_ref, cfbias_ref, fji_ref):
        dot = functools.partial(jnp.dot, preferred_element_type=jnp.float32)
        ea = _b16(ea_ref[...])
        d1, d2 = _d12(sp_ref[...], to_ref[...], sw1_ref, sw2_ref, tw1_ref,
                      tw2_ref)
        f = _silu(dot(ea, fpw_ref[...]) + fpb_ref[...])
        fji_ref[...] = f[:, :H] + _silu(
            dot(_b16(f[:, H:2 * H] * d1), cfa_ref[...])
            + dot(_b16(f[:, 2 * H:] * d2), cfb_ref[...])
            + cfbias_ref[...]
        )

    return pl.pallas_call(
        body,
        grid=(E // BE,),
        in_specs=[
            pl.BlockSpec((BE, H), lambda i: (i, 0)),
            pl.BlockSpec((BE, sphe_d), lambda i: (i, 0)),
            pl.BlockSpec((BE, tors_d), lambda i: (i, 0)),
            _full((sphe_d, mid)), _full((mid, H)),
            _full((tors_d, mid)), _full((mid, H)),
            _full((H, 3 * H)), _full((1, 3 * H)),
            _full((H, H)), _full((H, H)), _full((1, H)),
        ],
        out_specs=pl.BlockSpec((BE, H), lambda i: (i, 0)),
        out_shape=jax.ShapeDtypeStruct((E, H), jnp.float32),
    )(
        edge_attr, sphe16, tors16,
        _b16(p["sphe_w1"]), _b16(p["sphe_w2"]),
        _b16(p["tors_w1"]), _b16(p["tors_w2"]),
        _b16(p["f_proj_w"]), p["f_proj_b"].reshape(1, 3 * H),
        _b16(p["cat_f_w"][:H]), _b16(p["cat_f_w"][H:]),
        p["cat_f_b"].reshape(1, H),
    )


def _tc_pay(edge_attr, sphe16, tors16, dist2, v_g, p, chunk, n_chunks):
    E = edge_attr.shape[0]
    Eh = E // n_chunks
    nb = Eh // BE
    off = chunk * nb
    sphe_d, tors_d = sphe16.shape[1], tors16.shape[1]
    mid = p["sphe_w1"].shape[1]

    def body(ea_ref, sp_ref, to_ref, di_ref, vg_ref,
             sw1_ref, sw2_ref, tw1_ref, tw2_ref,
             dvw_ref, dvb_ref, smw_ref, smb_ref,
             cma_ref, cmb_ref, pay_ref):
        dot = functools.partial(jnp.dot, preferred_element_type=jnp.float32)
        ea = _b16(ea_ref[...])
        d1, d2 = _d12(sp_ref[...], to_ref[...], sw1_ref, sw2_ref, tw1_ref,
                      tw2_ref)
        r = di_ref[0]
        y = r * (jnp.pi / (2.0 * CUT))
        t = y * y
        c = 1.0 + t * (-0.5 + t * (0.041666668 + t * (-0.0013888889
            + t * (2.4801587e-5 + t * -2.7557319e-7))))
        cut2 = c * c
        nrow = BE // H
        e_div = lax.broadcasted_iota(jnp.int32, (BE, nrow), 0) // H
        k_idx = lax.broadcasted_iota(jnp.int32, (BE, nrow), 1)
        rep = dot((e_div == k_idx).astype(jnp.float32), cut2)
        e_mod = lax.broadcasted_iota(jnp.int32, (BE, H), 0) % H
        l_idx = lax.broadcasted_iota(jnp.int32, (BE, H), 1)
        keep = _b16(jnp.where(e_mod == l_idx, rep, 0.0))
        cut = dot(keep, jnp.ones((H, H), jnp.bfloat16))
        dv = _silu(dot(ea, dvw_ref[...]) + dvb_ref[...]) * cut
        v_j = vg_ref[...] * dv
        s = _silu(dot(_b16(v_j), smw_ref[...]) + smb_ref[...])
        pay_ref[0] = v_j
        pay_ref[1] = dot(_b16(s[:, :H] * d1), cma_ref[...]) + dot(
            _b16(s[:, H:] * d2), cmb_ref[...])

    return pl.pallas_call(
        body,
        grid=(nb,),
        in_specs=[
            pl.BlockSpec((BE, H), lambda i: (i + off, 0)),
            pl.BlockSpec((BE, sphe_d), lambda i: (i + off, 0)),
            pl.BlockSpec((BE, tors_d), lambda i: (i + off, 0)),
            pl.BlockSpec((1, BE // H, H), lambda i: (i + off, 0, 0)),
            pl.BlockSpec((BE, H), lambda i: (i + off, 0)),
            _full((sphe_d, mid)), _full((mid, H)),
            _full((tors_d, mid)), _full((mid, H)),
            _full((H, H)), _full((1, H)),
            _full((H, 2 * H)), _full((1, 2 * H)),
            _full((H, H)), _full((H, H)),
        ],
        out_specs=pl.BlockSpec((2, BE, H), lambda i: (0, i, 0)),
        out_shape=jax.ShapeDtypeStruct((2, Eh, H), jnp.float32),
    )(
        edge_attr, sphe16, tors16, dist2, v_g,
        _b16(p["sphe_w1"]), _b16(p["sphe_w2"]),
        _b16(p["tors_w1"]), _b16(p["tors_w2"]),
        _b16(p["dv_w"]), p["dv_b"].reshape(1, H),
        _b16(p["s_msg_w"]), p["s_msg_b"].reshape(1, 2 * H),
        _b16(p["cat_msg_w"][:H]), _b16(p["cat_msg_w"][H:]),
    )


def _tc_node_update(x_t, agg, p):
    n = x_t.shape[0]
    BN = 2000

    def body(xt_ref, agg_ref, omw_ref, omb_ref, fw_ref, fb_ref, o_ref):
        dot = functools.partial(jnp.dot, preferred_element_type=jnp.float32)
        s_msg = agg_ref[0]
        d_msg = agg_ref[1]
        om = dot(s_msg, omw_ref[...]) + omb_ref[...]
        pre = xt_ref[...] + om[:, :H] + om[:, H:] * d_msg
        o_ref[...] = _silu(dot(pre, fw_ref[...]) + fb_ref[...])

    return pl.pallas_call(
        body,
        grid=(n // BN,),
        in_specs=[
            pl.BlockSpec((BN, H), lambda i: (i, 0)),
            pl.BlockSpec((2, BN, H), lambda i: (0, i, 0)),
            pl.BlockSpec((H, 2 * H), lambda i: (0, 0)),
            pl.BlockSpec((1, 2 * H), lambda i: (0, 0)),
            pl.BlockSpec((H, H), lambda i: (0, 0)),
            pl.BlockSpec((1, H), lambda i: (0, 0)),
        ],
        out_specs=pl.BlockSpec((BN, H), lambda i: (i, 0)),
        out_shape=jax.ShapeDtypeStruct((n, H), jnp.float32),
    )(x_t, agg, p["o_msg_w"], p["o_msg_b"].reshape(1, 2 * H),
      p["final_w"], p["final_b"].reshape(1, H))


def kernel(x_s, x_t, edge_index, edge_attr, sphe_emb, torsion_emb, dist, params):
    n = x_s.shape[0]
    E = edge_attr.shape[0]
    n_chunks = 2
    Eh = E // n_chunks
    src = edge_index[0].astype(jnp.int32)
    dst = edge_index[1].astype(jnp.int32)
    sphe16 = _b16(sphe_emb)
    tors16 = _b16(torsion_emb)
    dist2 = dist.reshape(E // BE, BE // H, H)

    v = _tc_node_proj(x_s, params["node_w"], params["node_b"])
    v_g = _sc_gather(v, src)
    f_ji = _tc_fji(edge_attr, sphe16, tors16, params)
    agg = jnp.zeros((NC, n, H), jnp.float32)
    for k in range(n_chunks):
        pay = _tc_pay(edge_attr, sphe16, tors16, dist2, v_g, params,
                      k, n_chunks)
        agg = _sc_segsum(pay, lax.dynamic_slice(dst, (k * Eh,), (Eh,)), n, agg)
    h_t = _tc_node_update(x_t, agg, params)
    return (h_t, f_ji)


# speedup vs baseline: 1.0045x; 1.0045x over previous
import functools

import jax
import jax.numpy as jnp
from jax import lax
from jax.experimental import pallas as pl
from jax.experimental.pallas import tpu as pltpu
from jax.experimental.pallas import tpu_sc as plsc

H = 128
NC = 2
NS = 16
CUT = 8.0


def _silu(x):
    return 0.5 * x * (1.0 + jnp.tanh(0.5 * x))


def _b16(x):
    return x.astype(jnp.bfloat16)


def _sc_mesh():
    return plsc.VectorSubcoreMesh(
        core_axis_name="c", subcore_axis_name="s", num_cores=NC, num_subcores=NS
    )


def _sc_gather(table, idx):
    E = idx.shape[0]
    NW = NC * NS
    per_w = E // NW
    CH = 80
    n_ch = per_w // CH
    GPS = 5
    SUP = CH * GPS
    n_sup = per_w // SUP

    idx3 = idx.reshape(NW, n_ch, CH)

    @functools.partial(
        pl.kernel,
        out_type=jax.ShapeDtypeStruct((E, H), jnp.float32),
        mesh=_sc_mesh(),
        scratch_types=[
            pltpu.VMEM((n_ch, CH), jnp.int32),
            pltpu.VMEM((SUP, H), jnp.float32),
            pltpu.SemaphoreType.DMA,
            pltpu.SemaphoreType.DMA,
        ],
    )
    def k(table_hbm, idx_hbm, out_hbm, idx_v, rows_v, sem_i, sem_g):
        wid = lax.axis_index("s") * NC + lax.axis_index("c")
        base = wid * per_w
        pltpu.async_copy(idx_hbm.at[wid], idx_v, sem_i).wait()

        @pl.loop(0, n_sup)
        def _(j):
            descs = [
                pltpu.async_copy(
                    table_hbm.at[idx_v.at[j * GPS + kk]],
                    rows_v.at[pl.ds(kk * CH, CH)],
                    sem_g,
                )
                for kk in range(GPS)
            ]
            for d in descs:
                d.wait()
            pltpu.sync_copy(rows_v, out_hbm.at[pl.ds(base + j * SUP, SUP)])

    return k(table, idx3)


def _sc_segsum(payload, dst, n_nodes):
    E = dst.shape[0]
    per_s = E // NS
    CH = 80
    n_ch = per_s // CH
    GPS = 2
    SUP = CH * GPS
    n_sup = per_s // SUP
    n_pair = n_sup // 2
    rows_per_s = (n_nodes // NS) // 8 * 8
    tail_rows = n_nodes - (NS - 1) * rows_per_s - rows_per_s
    tail_off = NS * rows_per_s

    zeros = jnp.zeros((n_nodes, H), jnp.float32)
    dst4 = dst.reshape(NS, n_sup, GPS, CH)

    @functools.partial(
        pl.kernel,
        out_type=jax.ShapeDtypeStruct((NC, n_nodes, H), jnp.float32),
        mesh=_sc_mesh(),
        scratch_types=[
            pltpu.VMEM((GPS, CH), jnp.int32),
            pltpu.VMEM((GPS, CH), jnp.int32),
            pltpu.VMEM((SUP, H), jnp.float32),
            pltpu.VMEM((SUP, H), jnp.float32),
            pltpu.VMEM_SHARED((n_nodes, H), jnp.float32),
            pltpu.SemaphoreType.DMA,
            pltpu.SemaphoreType.DMA,
            pltpu.SemaphoreType.DMA,
            pltpu.SemaphoreType.DMA,
            pltpu.SemaphoreType.DMA,
        ],
    )
    def k(pay_hbm, dst_hbm, zero_hbm, out_hbm, ib0, ib1, rb0, rb1, acc_sh,
          si0, si1, sp0, sp1, sa):
        c = lax.axis_index("c")
        s = lax.axis_index("s")
        idxb, rowsb, sib, spb = (ib0, ib1), (rb0, rb1), (si0, si1), (sp0, sp1)
        base = s * per_s

        def start(j, b):
            pltpu.async_copy(dst_hbm.at[s, j], idxb[b], sib[b])
            pltpu.async_copy(
                pay_hbm.at[c, pl.ds(base + j * SUP, SUP)], rowsb[b], spb[b]
            )

        def work(b):
            pltpu.make_async_copy(dst_hbm.at[s, 0], idxb[b], sib[b]).wait()
            pltpu.make_async_copy(
                pay_hbm.at[c, pl.ds(base, SUP)], rowsb[b], spb[b]
            ).wait()
            descs = [
                pltpu.async_copy(
                    rowsb[b].at[pl.ds(kk * CH, CH)],
                    acc_sh.at[idxb[b].at[kk]],
                    sa,
                    add=True,
                )
                for kk in range(GPS)
            ]
            for d in descs:
                d.wait()

        start(0, 0)
        r0 = s * rows_per_s
        pltpu.sync_copy(
            zero_hbm.at[pl.ds(r0, rows_per_s)], acc_sh.at[pl.ds(r0, rows_per_s)]
        )

        @pl.when(s == NS - 1)
        def _():
            pltpu.sync_copy(
                zero_hbm.at[pl.ds(tail_off, tail_rows)],
                acc_sh.at[pl.ds(tail_off, tail_rows)],
            )

        plsc.subcore_barrier()

        @pl.loop(0, n_pair)
        def _(i):
            start(2 * i + 1, 1)
            work(0)
            start(2 * i + 2, 0)
            work(1)

        work(0)

        plsc.subcore_barrier()
        pltpu.sync_copy(
            acc_sh.at[pl.ds(r0, rows_per_s)], out_hbm.at[c, pl.ds(r0, rows_per_s)]
        )

        @pl.when(s == NS - 1)
        def _():
            pltpu.sync_copy(
                acc_sh.at[pl.ds(tail_off, tail_rows)],
                out_hbm.at[c, pl.ds(tail_off, tail_rows)],
            )

    return k(payload, dst4, zeros)


def _tc_node_proj(x_s, w, b):
    n = x_s.shape[0]
    BN = 2000

    def body(x_ref, w_ref, b_ref, o_ref):
        o_ref[...] = (
            jnp.dot(x_ref[...], w_ref[...], preferred_element_type=jnp.float32)
            + b_ref[...]
        )

    return pl.pallas_call(
        body,
        grid=(n // BN,),
        in_specs=[
            pl.BlockSpec((BN, H), lambda i: (i, 0)),
            pl.BlockSpec((H, H), lambda i: (0, 0)),
            pl.BlockSpec((1, H), lambda i: (0, 0)),
        ],
        out_specs=pl.BlockSpec((BN, H), lambda i: (i, 0)),
        out_shape=jax.ShapeDtypeStruct((n, H), jnp.float32),
    )(x_s, w, b.reshape(1, H))


def _tc_edge(edge_attr, sphe_emb, torsion_emb, dist, v_g, p):
    E = edge_attr.shape[0]
    BE = 2560
    sphe_d = sphe_emb.shape[1]
    tors_d = torsion_emb.shape[1]
    mid = p["sphe_w1"].shape[1]

    dist2 = dist.reshape(E // BE, BE // H, H)

    def body(ea_ref, sp_ref, to_ref, di_ref, vg_ref,
             sw1_ref, sw2_ref, tw1_ref, tw2_ref,
             dvw_ref, dvb_ref, smw_ref, smb_ref,
             cma_ref, cmb_ref,
             fpw_ref, fpb_ref, cfa_ref, cfb_ref, cfbias_ref,
             fji_ref, pay_ref):
        dot = functools.partial(jnp.dot, preferred_element_type=jnp.float32)
        ea = _b16(ea_ref[...])
        d1 = dot(_b16(dot(sp_ref[...], sw1_ref[...])), sw2_ref[...])
        d2 = dot(_b16(dot(to_ref[...], tw1_ref[...])), tw2_ref[...])
        r = di_ref[0]
        y = r * (jnp.pi / (2.0 * CUT))
        t = y * y
        c = 1.0 + t * (-0.5 + t * (0.041666668 + t * (-0.0013888889
            + t * (2.4801587e-5 + t * -2.7557319e-7))))
        cut2 = c * c
        nrow = BE // H
        e_div = lax.broadcasted_iota(jnp.int32, (BE, nrow), 0) // H
        k_idx = lax.broadcasted_iota(jnp.int32, (BE, nrow), 1)
        rep = dot((e_div == k_idx).astype(jnp.float32), cut2)
        e_mod = lax.broadcasted_iota(jnp.int32, (BE, H), 0) % H
        l_idx = lax.broadcasted_iota(jnp.int32, (BE, H), 1)
        keep = _b16(jnp.where(e_mod == l_idx, rep, 0.0))
        cut = dot(keep, jnp.ones((H, H), jnp.bfloat16))
        dv = _silu(dot(ea, dvw_ref[...]) + dvb_ref[...]) * cut
        v_j = vg_ref[...] * dv
        s = _silu(dot(_b16(v_j), smw_ref[...]) + smb_ref[...])
        pay_d = dot(_b16(s[:, :H] * d1), cma_ref[...]) + dot(
            _b16(s[:, H:] * d2), cmb_ref[...])
        f = _silu(dot(ea, fpw_ref[...]) + fpb_ref[...])
        f_ji = f[:, :H] + _silu(
            dot(_b16(f[:, H:2 * H] * d1), cfa_ref[...])
            + dot(_b16(f[:, 2 * H:] * d2), cfb_ref[...])
            + cfbias_ref[...]
        )
        fji_ref[...] = f_ji
        pay_ref[0] = v_j
        pay_ref[1] = pay_d

    def full(shape):
        return pl.BlockSpec(shape, lambda *_: tuple(0 for _ in shape))

    out = pl.pallas_call(
        body,
        grid=(E // BE,),
        in_specs=[
            pl.BlockSpec((BE, H), lambda i: (i, 0)),
            pl.BlockSpec((BE, sphe_d), lambda i: (i, 0)),
            pl.BlockSpec((BE, tors_d), lambda i: (i, 0)),
            pl.BlockSpec((1, BE // H, H), lambda i: (i, 0, 0)),
            pl.BlockSpec((BE, H), lambda i: (i, 0)),
            full((sphe_d, mid)), full((mid, H)),
            full((tors_d, mid)), full((mid, H)),
            full((H, H)), full((1, H)),
            full((H, 2 * H)), full((1, 2 * H)),
            full((H, H)), full((H, H)),
            full((H, 3 * H)), full((1, 3 * H)),
            full((H, H)), full((H, H)), full((1, H)),
        ],
        out_specs=[
            pl.BlockSpec((BE, H), lambda i: (i, 0)),
            pl.BlockSpec((2, BE, H), lambda i: (0, i, 0)),
        ],
        out_shape=[
            jax.ShapeDtypeStruct((E, H), jnp.float32),
            jax.ShapeDtypeStruct((2, E, H), jnp.float32),
        ],
    )(
        edge_attr, _b16(sphe_emb), _b16(torsion_emb), dist2, v_g,
        _b16(p["sphe_w1"]), _b16(p["sphe_w2"]),
        _b16(p["tors_w1"]), _b16(p["tors_w2"]),
        _b16(p["dv_w"]), p["dv_b"].reshape(1, H),
        _b16(p["s_msg_w"]), p["s_msg_b"].reshape(1, 2 * H),
        _b16(p["cat_msg_w"][:H]), _b16(p["cat_msg_w"][H:]),
        _b16(p["f_proj_w"]), p["f_proj_b"].reshape(1, 3 * H),
        _b16(p["cat_f_w"][:H]), _b16(p["cat_f_w"][H:]), p["cat_f_b"].reshape(1, H),
    )
    return out[0], out[1]


def _tc_node_update(x_t, agg, p):
    n = x_t.shape[0]
    BN = 2000

    def body(xt_ref, agg_ref, omw_ref, omb_ref, fw_ref, fb_ref, o_ref):
        dot = functools.partial(jnp.dot, preferred_element_type=jnp.float32)
        s_msg = agg_ref[0]
        d_msg = agg_ref[1]
        om = dot(s_msg, omw_ref[...]) + omb_ref[...]
        pre = xt_ref[...] + om[:, :H] + om[:, H:] * d_msg
        o_ref[...] = _silu(dot(pre, fw_ref[...]) + fb_ref[...])

    return pl.pallas_call(
        body,
        grid=(n // BN,),
        in_specs=[
            pl.BlockSpec((BN, H), lambda i: (i, 0)),
            pl.BlockSpec((2, BN, H), lambda i: (0, i, 0)),
            pl.BlockSpec((H, 2 * H), lambda i: (0, 0)),
            pl.BlockSpec((1, 2 * H), lambda i: (0, 0)),
            pl.BlockSpec((H, H), lambda i: (0, 0)),
            pl.BlockSpec((1, H), lambda i: (0, 0)),
        ],
        out_specs=pl.BlockSpec((BN, H), lambda i: (i, 0)),
        out_shape=jax.ShapeDtypeStruct((n, H), jnp.float32),
    )(x_t, agg, p["o_msg_w"], p["o_msg_b"].reshape(1, 2 * H),
      p["final_w"], p["final_b"].reshape(1, H))


def kernel(x_s, x_t, edge_index, edge_attr, sphe_emb, torsion_emb, dist, params):
    n = x_s.shape[0]
    src = edge_index[0].astype(jnp.int32)
    dst = edge_index[1].astype(jnp.int32)
    v = _tc_node_proj(x_s, params["node_w"], params["node_b"])
    v_g = _sc_gather(v, src)
    f_ji, pay = _tc_edge(edge_attr, sphe_emb, torsion_emb, dist, v_g, params)
    agg = _sc_segsum(pay, dst, n)
    h_t = _tc_node_update(x_t, agg, params)
    return (h_t, f_ji)
